# Initial kernel scaffold; baseline (speedup 1.0000x reference)
import jax
import jax.numpy as jnp
import numpy as np
from jax.experimental import pallas as pl


def kernel(points, anchors):
    n_points = points.shape[0]
    n_samples = anchors.shape[1]
    dist_bound = jnp.asarray(np.finfo(np.float32).max, dtype=jnp.float32)
    t_d = jnp.linalg.norm(points, axis=-1)
    t_n = jnp.where(t_d == 0, jnp.float32(1.0), t_d)
    t_p = points / t_n[:, None]
    cos = (t_p[:, 0:1] * anchors[0:1, :]
           + t_p[:, 1:2] * anchors[1:2, :]
           + t_p[:, 2:3] * anchors[2:3, :])
    angel_min_idx = jnp.argmax(cos, axis=-1)
    point_idx = jnp.arange(n_points)
    dist_grid = jnp.full((n_samples, n_points), dist_bound, dtype=jnp.float32)
    dist_grid = dist_grid.at[angel_min_idx, point_idx].set(t_d)
    dist_min_idx = jnp.argmin(dist_grid, axis=-1)
    dist_min = dist_grid[jnp.arange(n_samples), dist_min_idx]
    dist_min_idx = jnp.where(dist_min == dist_bound, -1, dist_min_idx)
    return dist_min_idx


# speedup vs baseline: 3.4482x; 3.4482x over previous
import jax
import jax.numpy as jnp
import numpy as np
from jax.experimental import pallas as pl
from jax.experimental.pallas import tpu as pltpu

_FMAX = np.float32(np.finfo(np.float32).max)
_P = 1024


def _rbf16(v):
    u = jax.lax.bitcast_convert_type(v, jnp.uint32)
    lsb = (u >> 16) & jnp.uint32(1)
    u = (u + jnp.uint32(0x7FFF) + lsb) & jnp.uint32(0xFFFF0000)
    return jax.lax.bitcast_convert_type(u, jnp.float32)


def _body(pts_ref, anch_ref, out_ref, accm_ref, acci_ref):
    i = pl.program_id(0)
    nsteps = pl.num_programs(0)
    P = pts_ref.shape[0]
    S = anch_ref.shape[1]

    pts = pts_ref[...]
    x = pts[:, 0:1]
    y = pts[:, 1:2]
    z = pts[:, 2:3]
    d = jnp.sqrt((x * x + y * y) + z * z)
    n = jnp.where(d == 0.0, jnp.float32(1.0), d)
    px = _rbf16(x / n)
    py = _rbf16(y / n)
    pz = _rbf16(z / n)

    anch = _rbf16(anch_ref[...])
    ax = anch[0:1, :]
    ay = anch[1:2, :]
    az = anch[2:3, :]

    cos = (px * ax + py * ay) + pz * az

    amax = jnp.max(cos, axis=1, keepdims=True)
    sidx = jax.lax.broadcasted_iota(jnp.int32, (P, S), 1)
    aidx = jnp.min(jnp.where(cos == amax, sidx, jnp.int32(S)),
                   axis=1, keepdims=True)
    dmask = jnp.where(sidx == aidx, d, _FMAX)
    bmin = jnp.min(dmask, axis=0, keepdims=True)
    pidx = jax.lax.broadcasted_iota(jnp.int32, (P, S), 0) + i * P
    bidx = jnp.min(jnp.where(dmask == bmin, pidx, jnp.int32(2**31 - 1)),
                   axis=0, keepdims=True)

    @pl.when(i == 0)
    def _():
        accm_ref[...] = jnp.full((1, S), _FMAX, jnp.float32)
        acci_ref[...] = jnp.zeros((1, S), jnp.int32)

    take = bmin < accm_ref[...]
    accm_ref[...] = jnp.where(take, bmin, accm_ref[...])
    acci_ref[...] = jnp.where(take, bidx, acci_ref[...])

    @pl.when(i == nsteps - 1)
    def _():
        out_ref[...] = jnp.where(accm_ref[...] == _FMAX, jnp.int32(-1),
                                 acci_ref[...])


def kernel(points, anchors):
    N = points.shape[0]
    S = anchors.shape[1]
    grid = (N // _P,)
    out = pl.pallas_call(
        _body,
        grid=grid,
        in_specs=[
            pl.BlockSpec((_P, 3), lambda i: (i, 0)),
            pl.BlockSpec((3, S), lambda i: (0, 0)),
        ],
        out_specs=pl.BlockSpec((1, S), lambda i: (0, 0)),
        out_shape=jax.ShapeDtypeStruct((1, S), jnp.int32),
        scratch_shapes=[
            pltpu.VMEM((1, S), jnp.float32),
            pltpu.VMEM((1, S), jnp.int32),
        ],
    )(points, anchors)
    return out.reshape(S)


# speedup vs baseline: 4.2550x; 1.2340x over previous
import jax
import jax.numpy as jnp
import numpy as np
from jax.experimental import pallas as pl
from jax.experimental.pallas import tpu as pltpu

_FMAX = np.float32(np.finfo(np.float32).max)
_P = 1024


def _rbf16(v):
    u = jax.lax.bitcast_convert_type(v, jnp.uint32)
    lsb = (u >> 16) & jnp.uint32(1)
    u = (u + jnp.uint32(0x7FFF) + lsb) & jnp.uint32(0xFFFF0000)
    return jax.lax.bitcast_convert_type(u, jnp.float32)


def _body(pts_ref, anch_ref, out_ref, accm_ref, acci_ref):
    i = pl.program_id(0)
    nsteps = pl.num_programs(0)
    P = pts_ref.shape[0]
    S = anch_ref.shape[1]

    pts = pts_ref[...]
    x = pts[:, 0:1]
    y = pts[:, 1:2]
    z = pts[:, 2:3]
    d = jnp.sqrt((x * x + y * y) + z * z)
    n = jnp.where(d == 0.0, jnp.float32(1.0), d)
    tp = (pts / n).astype(jnp.bfloat16)
    ab = anch_ref[...].astype(jnp.bfloat16)
    cos = jax.lax.dot_general(tp, ab, (((1,), (0,)), ((), ())),
                              preferred_element_type=jnp.float32)

    amax = jnp.max(cos, axis=1, keepdims=True)
    sidx = jax.lax.broadcasted_iota(jnp.int32, (P, S), 1)
    aidx = jnp.min(jnp.where(cos == amax, sidx, jnp.int32(S)),
                   axis=1, keepdims=True)
    dmask = jnp.where(sidx == aidx, d, _FMAX)
    bmin = jnp.min(dmask, axis=0, keepdims=True)
    pidx = jax.lax.broadcasted_iota(jnp.int32, (P, S), 0) + i * P
    bidx = jnp.min(jnp.where(dmask == bmin, pidx, jnp.int32(2**31 - 1)),
                   axis=0, keepdims=True)

    @pl.when(i == 0)
    def _():
        accm_ref[...] = jnp.full((1, S), _FMAX, jnp.float32)
        acci_ref[...] = jnp.zeros((1, S), jnp.int32)

    take = bmin < accm_ref[...]
    accm_ref[...] = jnp.where(take, bmin, accm_ref[...])
    acci_ref[...] = jnp.where(take, bidx, acci_ref[...])

    @pl.when(i == nsteps - 1)
    def _():
        out_ref[...] = jnp.where(accm_ref[...] == _FMAX, jnp.int32(-1),
                                 acci_ref[...])


def kernel(points, anchors):
    N = points.shape[0]
    S = anchors.shape[1]
    grid = (N // _P,)
    out = pl.pallas_call(
        _body,
        grid=grid,
        in_specs=[
            pl.BlockSpec((_P, 3), lambda i: (i, 0)),
            pl.BlockSpec((3, S), lambda i: (0, 0)),
        ],
        out_specs=pl.BlockSpec((1, S), lambda i: (0, 0)),
        out_shape=jax.ShapeDtypeStruct((1, S), jnp.int32),
        scratch_shapes=[
            pltpu.VMEM((1, S), jnp.float32),
            pltpu.VMEM((1, S), jnp.int32),
        ],
    )(points, anchors)
    return out.reshape(S)


# speedup vs baseline: 5.0950x; 1.1974x over previous
import jax
import jax.numpy as jnp
import numpy as np
from jax.experimental import pallas as pl
from jax.experimental.pallas import tpu as pltpu
from jax.experimental.pallas import tpu_sc as plsc

_FMAX = np.float32(np.finfo(np.float32).max)
_FMAXBITS = np.int32(np.float32(_FMAX).view(np.int32))
_IMAX = np.int32(2**31 - 1)
_P = 1024
_W = 32
_L = 16


def _assign_body(pts_ref, anch_ref, aidx_ref, dbits_ref):
    P = pts_ref.shape[0]
    S = anch_ref.shape[1]

    pts = pts_ref[...]
    x = pts[:, 0:1]
    y = pts[:, 1:2]
    z = pts[:, 2:3]
    d = jnp.sqrt((x * x + y * y) + z * z)
    n = jnp.where(d == 0.0, jnp.float32(1.0), d)
    tp = (pts / n).astype(jnp.bfloat16)
    ab = anch_ref[...].astype(jnp.bfloat16)
    cos = jax.lax.dot_general(tp, ab, (((1,), (0,)), ((), ())),
                              preferred_element_type=jnp.float32)

    amax = jnp.max(cos, axis=1, keepdims=True)
    sidx = jax.lax.broadcasted_iota(jnp.int32, (P, S), 1)
    aidx = jnp.min(jnp.where(cos == amax, sidx, jnp.int32(S)),
                   axis=1, keepdims=True)

    dbits = jax.lax.bitcast_convert_type(d, jnp.int32)
    aidx_ref[...] = aidx.reshape(P // 128, 128)
    dbits_ref[...] = dbits.reshape(P // 128, 128)


def _sc_body(aidx_hbm, dbits_hbm, pbd_hbm, pbi_hbm, av, dv, bd, bi):
    C = av.shape[0]
    S = pbd_hbm.shape[1] // _L
    wid = jax.lax.axis_index("c") * 16 + jax.lax.axis_index("s")
    pltpu.sync_copy(aidx_hbm.at[pl.ds(wid * C, C)], av)
    pltpu.sync_copy(dbits_hbm.at[pl.ds(wid * C, C)], dv)

    fmax16 = jnp.full((_L,), _FMAXBITS, jnp.int32)

    def init_step(j, carry):
        bd[pl.ds(j * _L, _L)] = fmax16
        return carry

    jax.lax.fori_loop(0, bd.shape[0] // _L, init_step, 0)

    lane = jax.lax.iota(jnp.int32, _L)
    slot0 = lane * S

    def step(g, carry):
        a = av[pl.ds(g * _L, _L)]
        db = dv[pl.ds(g * _L, _L)]
        slot = slot0 + a
        cur = plsc.load_gather(bd, [slot])
        m = db < cur
        plsc.store_scatter(bd, [slot], db, mask=m)
        gidx = wid * C + g * _L + lane
        plsc.store_scatter(bi, [slot], gidx, mask=m)
        return carry

    jax.lax.fori_loop(0, C // _L, step, 0)

    pltpu.sync_copy(bd, pbd_hbm.at[wid])
    pltpu.sync_copy(bi, pbi_hbm.at[wid])


def _combine_body(pbd_ref, pbi_ref, out_ref):
    pbd = pbd_ref[...]
    pbi = pbi_ref[...]
    accd = jnp.min(pbd, axis=0, keepdims=True)
    eq = pbd == accd
    bi_sel = jnp.min(jnp.where(eq, pbi, _IMAX),
                     axis=0, keepdims=True)
    out_ref[...] = jnp.where(accd == _FMAXBITS, jnp.int32(-1), bi_sel)


def kernel(points, anchors):
    N = points.shape[0]
    S = anchors.shape[1]
    aidx, dbits = pl.pallas_call(
        _assign_body,
        grid=(N // _P,),
        in_specs=[
            pl.BlockSpec((_P, 3), lambda i: (i, 0)),
            pl.BlockSpec((3, S), lambda i: (0, 0)),
        ],
        out_specs=[
            pl.BlockSpec((_P // 128, 128), lambda i: (i, 0)),
            pl.BlockSpec((_P // 128, 128), lambda i: (i, 0)),
        ],
        out_shape=[
            jax.ShapeDtypeStruct((N // 128, 128), jnp.int32),
            jax.ShapeDtypeStruct((N // 128, 128), jnp.int32),
        ],
    )(points, anchors)

    C = N // _W
    mesh = plsc.VectorSubcoreMesh(core_axis_name="c", subcore_axis_name="s")
    pbd, pbi = pl.kernel(
        _sc_body,
        out_type=[
            jax.ShapeDtypeStruct((_W, _L * S), jnp.int32),
            jax.ShapeDtypeStruct((_W, _L * S), jnp.int32),
        ],
        mesh=mesh,
        compiler_params=pltpu.CompilerParams(needs_layout_passes=False),
        scratch_types=[
            pltpu.VMEM((C,), jnp.int32),
            pltpu.VMEM((C,), jnp.int32),
            pltpu.VMEM((_L * S,), jnp.int32),
            pltpu.VMEM((_L * S,), jnp.int32),
        ],
    )(aidx.reshape(N), dbits.reshape(N))

    out = pl.pallas_call(
        _combine_body,
        out_shape=jax.ShapeDtypeStruct((1, S), jnp.int32),
    )(pbd.reshape(_W * _L, S), pbi.reshape(_W * _L, S))
    return out.reshape(S)
